# baseline (speedup 1.0000x reference)
import functools

import jax
import jax.numpy as jnp
from jax import lax
from jax.experimental import pallas as pl
from jax.experimental.pallas import tpu as pltpu
from jax.experimental.pallas import tpu_sc as plsc

VOCAB = 100000
D = 128
SEQ = 200
BATCH = 1024
N = BATCH * SEQ
EPS = 1e-5

NC = 2
NS = 16
NW = NC * NS

B_PER_W = N // NW
CHUNK = 256
NCH = B_PER_W // CHUNK
GBUF = 2
HMASK = -65536


@functools.cache
def _make_sc_gather():
    mesh = plsc.VectorSubcoreMesh(core_axis_name="c", subcore_axis_name="s")

    @functools.partial(
        pl.kernel,
        mesh=mesh,
        out_type=jax.ShapeDtypeStruct((N // 2, D), jnp.int32),
        scratch_types=[
            pltpu.VMEM((B_PER_W,), jnp.int32),
        ] + [pltpu.VMEM((CHUNK, D), jnp.int32)] * GBUF
          + [pltpu.VMEM((CHUNK // 2, D), jnp.int32)] * 2
          + [pltpu.SemaphoreType.DMA] * (GBUF + 2),
    )
    def gather_k(idx_hbm, table_hbm, out_hbm, idx_v, *rest):
        gbufs = rest[:GBUF]
        pbufs = rest[GBUF:GBUF + 2]
        gsem = rest[GBUF + 2:2 * GBUF + 2]
        wsem = rest[2 * GBUF + 2:]
        wid = lax.axis_index("s") * NC + lax.axis_index("c")
        base = wid * B_PER_W
        base2 = wid * (B_PER_W // 2)
        pltpu.sync_copy(idx_hbm.at[pl.ds(base, B_PER_W)], idx_v)

        def start_gather(c):
            i = c % GBUF
            return pltpu.async_copy(
                table_hbm.at[idx_v.at[pl.ds(c * CHUNK, CHUNK)]],
                gbufs[i], gsem[i])

        def pack_chunk(src, dst):
            def body(k, carry):
                for j in range(D // 16):
                    sl = pl.ds(16 * j, 16)
                    wa = src[2 * k, sl]
                    wb = src[2 * k + 1, sl]
                    dst[k, sl] = jnp.bitwise_or(
                        jnp.bitwise_and(wb, jnp.int32(HMASK)),
                        lax.shift_right_logical(wa, 16))
                return carry
            lax.fori_loop(0, CHUNK // 2, body, 0)

        def start_write(c, i):
            return pltpu.async_copy(
                pbufs[i],
                out_hbm.at[pl.ds(base2 + c * (CHUNK // 2), CHUNK // 2)],
                wsem[i])

        g = [None] * GBUF
        w = [None, None]
        g[0] = start_gather(0)
        for c in range(NCH):
            if c + 1 < NCH:
                g[(c + 1) % GBUF] = start_gather(c + 1)
            g[c % GBUF].wait()
            pb = c % 2
            if w[pb] is not None:
                w[pb].wait()
            pack_chunk(gbufs[c % GBUF], pbufs[pb])
            w[pb] = start_write(c, pb)
        for h in w:
            if h is not None:
                h.wait()

    return gather_k


_BB = 64
_SH = SEQ // 2


def _ln_body(w_ref, tte_ref, tto_ref, pose_ref, poso_ref, seg_ref,
             g_ref, b_ref, out_ref):
    w = w_ref[...]
    a = lax.bitcast_convert_type(lax.shift_left(w, 16), jnp.float32)
    b = lax.bitcast_convert_type(jnp.bitwise_and(w, jnp.int32(HMASK)), jnp.float32)
    a3 = a.reshape(_BB, _SH, D)
    b3 = b.reshape(_BB, _SH, D)
    seg = seg_ref[...]
    gam = g_ref[...]
    bet = b_ref[...]
    ones = jnp.full((D, D), 1.0 / D, jnp.float32)
    dims = (((1,), (0,)), ((), ()))

    def ln(x3, tt, pos):
        segv = jnp.where((tt[..., None] == 0), seg[0][None, None, :],
                         seg[1][None, None, :])
        emb = (x3 + pos[None, :, :] + segv).reshape(-1, D)
        mean = lax.dot_general(emb, ones, dims, preferred_element_type=jnp.float32)
        msq = lax.dot_general(emb * emb, ones, dims, preferred_element_type=jnp.float32)
        var = msq - mean * mean
        outm = (emb - mean) * (lax.rsqrt(var + EPS) * gam) + bet
        return outm.reshape(_BB, _SH, 1, D)

    oe = ln(a3, tte_ref[...], pose_ref[...])
    oo = ln(b3, tto_ref[...], poso_ref[...])
    out_ref[...] = jnp.concatenate([oe, oo], axis=2).reshape(_BB, SEQ, D)


def _tc_layernorm(w2, tte, tto, pose, poso, seg, gamma, beta):
    return pl.pallas_call(
        _ln_body,
        grid=(BATCH // _BB,),
        in_specs=[
            pl.BlockSpec((_BB * _SH, D), lambda i: (i, 0)),
            pl.BlockSpec((_BB, _SH), lambda i: (i, 0)),
            pl.BlockSpec((_BB, _SH), lambda i: (i, 0)),
            pl.BlockSpec((_SH, D), lambda i: (0, 0)),
            pl.BlockSpec((_SH, D), lambda i: (0, 0)),
            pl.BlockSpec((2, D), lambda i: (0, 0)),
            pl.BlockSpec((1, D), lambda i: (0, 0)),
            pl.BlockSpec((1, D), lambda i: (0, 0)),
        ],
        out_specs=pl.BlockSpec((_BB, SEQ, D), lambda i: (i, 0, 0)),
        out_shape=jax.ShapeDtypeStruct((BATCH, SEQ, D), jnp.float32),
    )(w2, tte, tto, pose, poso, seg, gamma, beta)


def kernel(input_ids, token_type_ids, token_table, position_table, segment_table, gamma, beta):
    ids = input_ids.astype(jnp.int32)
    tt = token_type_ids.astype(jnp.int32)
    w2 = _make_sc_gather()(
        ids.reshape(-1), lax.bitcast_convert_type(token_table, jnp.int32))
    return _tc_layernorm(
        w2,
        tt[:, 0::2],
        tt[:, 1::2],
        position_table[0:SEQ:2],
        position_table[1:SEQ:2],
        segment_table,
        gamma.reshape(1, D),
        beta.reshape(1, D),
    )


# speedup vs baseline: 1.2569x; 1.2569x over previous
import functools

import jax
import jax.numpy as jnp
from jax import lax
from jax.experimental import pallas as pl
from jax.experimental.pallas import tpu as pltpu
from jax.experimental.pallas import tpu_sc as plsc

VOCAB = 100000
D = 128
SEQ = 200
BATCH = 1024
N = BATCH * SEQ
EPS = 1e-5

NC = 2
NS = 16
NW = NC * NS

B_PER_W = N // NW
CHUNK = 256
NCH = B_PER_W // CHUNK
GBUF = 2
HMASK = -65536


@functools.cache
def _make_sc_gather():
    mesh = plsc.VectorSubcoreMesh(core_axis_name="c", subcore_axis_name="s")

    @functools.partial(
        pl.kernel,
        mesh=mesh,
        out_type=jax.ShapeDtypeStruct((N // 2, D), jnp.int32),
        scratch_types=[
            pltpu.VMEM((B_PER_W,), jnp.int32),
        ] + [pltpu.VMEM((CHUNK, D), jnp.int32)] * GBUF
          + [pltpu.VMEM((CHUNK // 2, D), jnp.int32)] * 2
          + [pltpu.SemaphoreType.DMA] * (GBUF + 2),
    )
    def gather_k(idx_hbm, table_hbm, out_hbm, idx_v, *rest):
        gbufs = rest[:GBUF]
        pbufs = rest[GBUF:GBUF + 2]
        gsem = rest[GBUF + 2:2 * GBUF + 2]
        wsem = rest[2 * GBUF + 2:]
        wid = lax.axis_index("s") * NC + lax.axis_index("c")
        base = wid * B_PER_W
        base2 = wid * (B_PER_W // 2)
        pltpu.sync_copy(idx_hbm.at[pl.ds(base, B_PER_W)], idx_v)

        def start_gather(c):
            i = c % GBUF
            return pltpu.async_copy(
                table_hbm.at[idx_v.at[pl.ds(c * CHUNK, CHUNK)]],
                gbufs[i], gsem[i])

        def pack_chunk(src, dst):
            @plsc.parallel_loop(0, CHUNK // 2, unroll=4)
            def _body(k):
                for j in range(D // 16):
                    sl = pl.ds(16 * j, 16)
                    wa = src[2 * k, sl]
                    wb = src[2 * k + 1, sl]
                    dst[k, sl] = jnp.bitwise_or(
                        jnp.bitwise_and(wb, jnp.int32(HMASK)),
                        lax.shift_right_logical(wa, 16))

        def start_write(c, i):
            return pltpu.async_copy(
                pbufs[i],
                out_hbm.at[pl.ds(base2 + c * (CHUNK // 2), CHUNK // 2)],
                wsem[i])

        g = [None] * GBUF
        w = [None, None]
        g[0] = start_gather(0)
        for c in range(NCH):
            if c + 1 < NCH:
                g[(c + 1) % GBUF] = start_gather(c + 1)
            g[c % GBUF].wait()
            pb = c % 2
            if w[pb] is not None:
                w[pb].wait()
            pack_chunk(gbufs[c % GBUF], pbufs[pb])
            w[pb] = start_write(c, pb)
        for h in w:
            if h is not None:
                h.wait()

    return gather_k


_BB = 64
_SH = SEQ // 2


def _ln_body(w_ref, tte_ref, tto_ref, pose_ref, poso_ref, seg_ref,
             g_ref, b_ref, out_ref):
    w = w_ref[...]
    a = lax.bitcast_convert_type(lax.shift_left(w, 16), jnp.float32)
    b = lax.bitcast_convert_type(jnp.bitwise_and(w, jnp.int32(HMASK)), jnp.float32)
    a3 = a.reshape(_BB, _SH, D)
    b3 = b.reshape(_BB, _SH, D)
    seg = seg_ref[...]
    gam = g_ref[...]
    bet = b_ref[...]
    ones = jnp.full((D, D), 1.0 / D, jnp.float32)
    dims = (((1,), (0,)), ((), ()))

    def ln(x3, tt, pos):
        segv = jnp.where((tt[..., None] == 0), seg[0][None, None, :],
                         seg[1][None, None, :])
        emb = (x3 + pos[None, :, :] + segv).reshape(-1, D)
        mean = lax.dot_general(emb, ones, dims, preferred_element_type=jnp.float32)
        msq = lax.dot_general(emb * emb, ones, dims, preferred_element_type=jnp.float32)
        var = msq - mean * mean
        outm = (emb - mean) * (lax.rsqrt(var + EPS) * gam) + bet
        return outm.reshape(_BB, _SH, 1, D)

    oe = ln(a3, tte_ref[...], pose_ref[...])
    oo = ln(b3, tto_ref[...], poso_ref[...])
    out_ref[...] = jnp.concatenate([oe, oo], axis=2).reshape(_BB, SEQ, D)


def _tc_layernorm(w2, tte, tto, pose, poso, seg, gamma, beta):
    return pl.pallas_call(
        _ln_body,
        grid=(BATCH // _BB,),
        in_specs=[
            pl.BlockSpec((_BB * _SH, D), lambda i: (i, 0)),
            pl.BlockSpec((_BB, _SH), lambda i: (i, 0)),
            pl.BlockSpec((_BB, _SH), lambda i: (i, 0)),
            pl.BlockSpec((_SH, D), lambda i: (0, 0)),
            pl.BlockSpec((_SH, D), lambda i: (0, 0)),
            pl.BlockSpec((2, D), lambda i: (0, 0)),
            pl.BlockSpec((1, D), lambda i: (0, 0)),
            pl.BlockSpec((1, D), lambda i: (0, 0)),
        ],
        out_specs=pl.BlockSpec((_BB, SEQ, D), lambda i: (i, 0, 0)),
        out_shape=jax.ShapeDtypeStruct((BATCH, SEQ, D), jnp.float32),
    )(w2, tte, tto, pose, poso, seg, gamma, beta)


def kernel(input_ids, token_type_ids, token_table, position_table, segment_table, gamma, beta):
    ids = input_ids.astype(jnp.int32)
    tt = token_type_ids.astype(jnp.int32)
    w2 = _make_sc_gather()(
        ids.reshape(-1), lax.bitcast_convert_type(token_table, jnp.int32))
    return _tc_layernorm(
        w2,
        tt[:, 0::2],
        tt[:, 1::2],
        position_table[0:SEQ:2],
        position_table[1:SEQ:2],
        segment_table,
        gamma.reshape(1, D),
        beta.reshape(1, D),
    )


# speedup vs baseline: 2.2577x; 1.7963x over previous
import functools

import jax
import jax.numpy as jnp
from jax import lax
from jax.experimental import pallas as pl
from jax.experimental.pallas import tpu as pltpu
from jax.experimental.pallas import tpu_sc as plsc

VOCAB = 100000
D = 128
SEQ = 200
BATCH = 1024
EPS = 1e-5

NC = 2
NS = 16
NW = NC * NS

PIECES = 1
PB = BATCH // PIECES
NP = PB * SEQ
B_PER_W = NP // NW
CHUNK = 200
NCH = B_PER_W // CHUNK
NBUF = 4


@functools.cache
def _make_sc_gather():
    mesh = plsc.VectorSubcoreMesh(core_axis_name="c", subcore_axis_name="s")

    @functools.partial(
        pl.kernel,
        mesh=mesh,
        out_type=jax.ShapeDtypeStruct((NP, D), jnp.float32),
        scratch_types=[
            pltpu.VMEM((B_PER_W,), jnp.int32),
        ] + [pltpu.VMEM((CHUNK, D), jnp.float32)] * NBUF
          + [pltpu.SemaphoreType.DMA] * (2 * NBUF),
    )
    def gather_k(idx_hbm, table_hbm, out_hbm, idx_v, *bufs_and_sems):
        bufs = bufs_and_sems[:NBUF]
        gsem = bufs_and_sems[NBUF:2 * NBUF]
        wsem = bufs_and_sems[2 * NBUF:]
        wid = lax.axis_index("s") * NC + lax.axis_index("c")
        base = wid * B_PER_W
        pltpu.sync_copy(idx_hbm.at[pl.ds(base, B_PER_W)], idx_v)

        def start_gather(c, which):
            return pltpu.async_copy(
                table_hbm.at[idx_v.at[pl.ds(c * CHUNK, CHUNK)]],
                bufs[which], gsem[which])

        def start_write(c, which):
            return pltpu.async_copy(
                bufs[which], out_hbm.at[pl.ds(base + c * CHUNK, CHUNK)],
                wsem[which])

        g = [None] * NBUF
        w = [None] * NBUF
        lead = NBUF // 2
        for c in range(min(lead, NCH)):
            g[c % NBUF] = start_gather(c, c % NBUF)
        for c in range(NCH):
            cur = c % NBUF
            if c + lead < NCH:
                b2 = (c + lead) % NBUF
                if w[b2] is not None:
                    w[b2].wait()
                    w[b2] = None
                g[b2] = start_gather(c + lead, b2)
            g[cur].wait()
            if w[cur] is not None:
                w[cur].wait()
            w[cur] = start_write(c, cur)
        for h in w:
            if h is not None:
                h.wait()

    return gather_k


def _ln_body(tok_ref, tt_ref, pos_ref, seg_ref, g_ref, b_ref, out_ref):
    tok = tok_ref[...]
    tt = tt_ref[...]
    pos = pos_ref[...]
    seg = seg_ref[...]
    segv = jnp.where((tt[..., None] == 0), seg[0][None, None, :], seg[1][None, None, :])
    emb = (tok + pos[None, :, :] + segv).reshape(-1, D)
    ones = jnp.full((D, D), 1.0 / D, jnp.float32)
    dims = (((1,), (0,)), ((), ()))
    mean = lax.dot_general(emb, ones, dims, preferred_element_type=jnp.float32)
    msq = lax.dot_general(emb * emb, ones, dims, preferred_element_type=jnp.float32)
    var = msq - mean * mean
    rinv = lax.rsqrt(var + EPS)
    outm = (emb - mean) * (rinv * g_ref[...]) + b_ref[...]
    out_ref[...] = outm.reshape(tok.shape)


_BB = 128


def _tc_layernorm(tok, tt, pos, seg, gamma, beta):
    return pl.pallas_call(
        _ln_body,
        grid=(PB // _BB,),
        in_specs=[
            pl.BlockSpec((_BB, SEQ, D), lambda i: (i, 0, 0)),
            pl.BlockSpec((_BB, SEQ), lambda i: (i, 0)),
            pl.BlockSpec((SEQ, D), lambda i: (0, 0)),
            pl.BlockSpec((2, D), lambda i: (0, 0)),
            pl.BlockSpec((1, D), lambda i: (0, 0)),
            pl.BlockSpec((1, D), lambda i: (0, 0)),
        ],
        out_specs=pl.BlockSpec((_BB, SEQ, D), lambda i: (i, 0, 0)),
        out_shape=jax.ShapeDtypeStruct((PB, SEQ, D), jnp.float32),
    )(tok, tt, pos, seg, gamma, beta)


def kernel(input_ids, token_type_ids, token_table, position_table, segment_table, gamma, beta):
    ids = input_ids.astype(jnp.int32)
    tt = token_type_ids.astype(jnp.int32)
    pos = position_table[:SEQ]
    g = gamma.reshape(1, D)
    b = beta.reshape(1, D)
    gather = _make_sc_gather()
    toks = [
        gather(ids[p * PB:(p + 1) * PB].reshape(-1), token_table).reshape(PB, SEQ, D)
        for p in range(PIECES)
    ]
    outs = [
        _tc_layernorm(toks[p], tt[p * PB:(p + 1) * PB], pos, segment_table, g, b)
        for p in range(PIECES)
    ]
    return jnp.concatenate(outs, axis=0)
